# baseline (speedup 1.0000x reference)
import jax
import jax.numpy as jnp
from jax import lax
from jax.experimental import pallas as pl
from jax.experimental.pallas import tpu as pltpu
from jax.experimental.pallas import tpu_sc as plsc

D_MODEL = 64
SCALE = 8.0
NUM_WORKERS = 32
SEQ = 50
N_B = 16384
P_B = 256
N_C = N_B // P_B


def _emb_body(xt_hbm, lut_hbm, out_hbm, idx2, rows, tbuf, gsem):
    wid = lax.axis_index("s") * 2 + lax.axis_index("c")
    lane = lax.iota(jnp.int32, 16)

    def item(t, carry):
        s = t // 2
        c = 2 * wid + t % 2
        pltpu.sync_copy(xt_hbm.at[s, c], idx2)
        cp0 = pltpu.async_copy(lut_hbm.at[idx2.at[0]],
                               rows.at[pl.ds(0, 128)], gsem)
        cp1 = pltpu.async_copy(lut_hbm.at[idx2.at[1]],
                               rows.at[pl.ds(128, 128)], gsem)
        cp0.wait()
        cp1.wait()

        def col(d, c2):
            dv = jnp.full((16,), d, jnp.int32)
            for j in range(P_B // 16):
                v = plsc.load_gather(rows, [lane + 16 * j, dv])
                tbuf[d, pl.ds(16 * j, 16)] = v * SCALE
            return c2

        lax.fori_loop(0, D_MODEL, col, 0)
        pltpu.sync_copy(tbuf, out_hbm.at[s, :, pl.ds(c * P_B, P_B)])
        return carry

    lax.fori_loop(0, SEQ * 2, item, 0)


def kernel(x, lut):
    B, S = x.shape
    xt = x.T.astype(jnp.int32).reshape(S, N_C, 2, 128)
    mesh = plsc.VectorSubcoreMesh(core_axis_name="c", subcore_axis_name="s")
    out = pl.kernel(
        _emb_body,
        mesh=mesh,
        out_type=jax.ShapeDtypeStruct((SEQ, D_MODEL, N_B), jnp.float32),
        scratch_types=[
            pltpu.VMEM((2, 128), jnp.int32),
            pltpu.VMEM((P_B, D_MODEL), jnp.float32),
            pltpu.VMEM((D_MODEL, P_B), jnp.float32),
            pltpu.SemaphoreType.DMA,
        ],
        compiler_params=pltpu.CompilerParams(
            use_tc_tiling_on_sc=False, needs_layout_passes=False),
    )(xt, lut)
    return jnp.transpose(out, (2, 0, 1))


# speedup vs baseline: 1.5826x; 1.5826x over previous
import jax
import jax.numpy as jnp
from jax import lax
from jax.experimental import pallas as pl
from jax.experimental.pallas import tpu as pltpu
from jax.experimental.pallas import tpu_sc as plsc

D_MODEL = 64
SCALE = 8.0
SEQ = 50
N_B = 16384
P_B = 256
N_C = N_B // P_B
N_ITEMS = SEQ * 2


def _emb_body(xt_hbm, lut_hbm, out_hbm, idxs, rowss, tbufs,
              isems, gsems, wsems):
    wid = lax.axis_index("s") * 2 + lax.axis_index("c")
    lane = lax.iota(jnp.int32, 16)
    dcols = [lane + 16 * k for k in range(4)]

    def idx_start(t, b):
        pltpu.async_copy(xt_hbm.at[t // 2, 2 * wid + t % 2],
                         idxs[b], isems[b])

    def idx_wait(b):
        pltpu.make_async_copy(xt_hbm.at[0, 0], idxs[b], isems[b]).wait()

    def gather_start(b):
        pltpu.async_copy(lut_hbm.at[idxs[b].at[0]],
                         rowss[b].at[pl.ds(0, 128)], gsems[b])
        pltpu.async_copy(lut_hbm.at[idxs[b].at[1]],
                         rowss[b].at[pl.ds(128, 128)], gsems[b])

    def gather_wait(b):
        for k in range(2):
            pltpu.make_async_copy(lut_hbm.at[idxs[b].at[0]],
                                  rowss[b].at[pl.ds(0, 128)],
                                  gsems[b]).wait()

    def transpose(b):
        rows, tb = rowss[b], tbufs[b]

        @plsc.parallel_loop(0, P_B, step=1, unroll=4)
        def _(r):
            rb = jnp.full((16,), r, jnp.int32)
            for k in range(4):
                v = rows[r, pl.ds(16 * k, 16)] * SCALE
                plsc.store_scatter(tb, [dcols[k], rb], v)

    def write_start(t, b):
        c = 2 * wid + t % 2
        pltpu.async_copy(tbufs[b],
                         out_hbm.at[t // 2, :, pl.ds(c * P_B, P_B)],
                         wsems[b])

    def write_wait(b):
        pltpu.make_async_copy(tbufs[b],
                              out_hbm.at[0, :, pl.ds(0, P_B)],
                              wsems[b]).wait()

    for b in range(2):
        idx_start(b, b)
    for b in range(2):
        idx_wait(b)
        gather_start(b)

    for b in range(2):
        gather_wait(b)
        idx_start(b + 2, b)
        transpose(b)
        idx_wait(b)
        gather_start(b)
        write_start(b, b)

    def group(g, carry):
        for b in range(2):
            t = 2 * g + b
            gather_wait(b)
            idx_start(t + 2, b)
            write_wait(b)
            transpose(b)
            idx_wait(b)
            gather_start(b)
            write_start(t, b)
        return carry

    lax.fori_loop(1, N_ITEMS // 2 - 1, group, 0)

    for b in range(2):
        t = N_ITEMS - 2 + b
        gather_wait(b)
        write_wait(b)
        transpose(b)
        write_start(t, b)
    for b in range(2):
        write_wait(b)


def kernel(x, lut):
    B, S = x.shape
    xt = x.T.astype(jnp.int32).reshape(S, N_C, 2, 128)
    mesh = plsc.VectorSubcoreMesh(core_axis_name="c", subcore_axis_name="s")
    out = pl.kernel(
        _emb_body,
        mesh=mesh,
        out_type=jax.ShapeDtypeStruct((SEQ, D_MODEL, N_B), jnp.float32),
        scratch_types=[
            [pltpu.VMEM((2, 128), jnp.int32) for _ in range(2)],
            [pltpu.VMEM((P_B, D_MODEL), jnp.float32) for _ in range(2)],
            [pltpu.VMEM((D_MODEL, P_B), jnp.float32) for _ in range(2)],
            [pltpu.SemaphoreType.DMA for _ in range(2)],
            [pltpu.SemaphoreType.DMA for _ in range(2)],
            [pltpu.SemaphoreType.DMA for _ in range(2)],
        ],
        compiler_params=pltpu.CompilerParams(
            use_tc_tiling_on_sc=False, needs_layout_passes=False),
    )(xt, lut)
    return jnp.transpose(out, (2, 0, 1))
